# Initial kernel scaffold; baseline (speedup 1.0000x reference)
import jax
import jax.numpy as jnp
from jax.experimental import pallas as pl


def kernel(atom_features, bond_features, connectivity, bond_transform, gru_kernel, gru_recurrent_kernel, gru_bias):
    raise NotImplementedError("write your pallas kernel here")


# speedup vs baseline: 3.1027x; 3.1027x over previous
import jax
import jax.numpy as jnp
from jax.experimental import pallas as pl
from jax.experimental.pallas import tpu as pltpu

ATOM_DIM = 64
BOND_DIM = 16
B, N, E = 4, 512, 4096
EC = 1024


def _scatter_body(atom_ref, src_ref, tgt_ref, bond_ref, agg_ref):
    a_sum = jnp.sum(atom_ref[...], axis=1, keepdims=True)
    acc = jnp.zeros((N, BOND_DIM), dtype=jnp.float32)
    for c in range(E // EC):
        src_c = src_ref[pl.ds(c * EC, EC), :]
        tgt_c = tgt_ref[:, pl.ds(c * EC, EC)]
        bond_c = bond_ref[pl.ds(c * EC, EC), :].astype(
            jnp.bfloat16).astype(jnp.float32)
        oh_src = (src_c == jax.lax.broadcasted_iota(jnp.int32, (EC, N), 1))
        s_col = jnp.dot(oh_src.astype(jnp.float32), a_sum,
                        preferred_element_type=jnp.float32, precision=jax.lax.Precision.HIGHEST)
        sbond = s_col * bond_c
        oh_tgt = (jax.lax.broadcasted_iota(jnp.int32, (N, EC), 0) == tgt_c)
        acc = acc + jnp.dot(oh_tgt.astype(jnp.float32), sbond,
                            preferred_element_type=jnp.float32, precision=jax.lax.Precision.HIGHEST)
    agg_ref[...] = acc


def _gru_body(atom_t_ref, agg_t_ref, bt_ref,
              kz_ref, kr_ref, kh_ref, kbz_ref, kbr_ref, kbh_ref,
              rz_ref, rr_ref, rh_ref,
              bz_ref, br_ref, bxh_ref, brh_ref,
              out_ref, xz_ref, xr_ref, xh_ref):
    t_mat = jnp.sum(bt_ref[...].astype(jnp.bfloat16).astype(jnp.float32),
                    axis=2)
    atom_t = atom_t_ref[...]
    agg64 = jnp.dot(agg_t_ref[...], t_mat,
                    preferred_element_type=jnp.float32,
                    precision=jax.lax.Precision.HIGHEST)

    def xpart(k_ref, kb_ref, b_ref):
        return (jnp.dot(atom_t, k_ref[...], preferred_element_type=jnp.float32)
                + jnp.dot(agg64, kb_ref[...],
                          preferred_element_type=jnp.float32)
                + b_ref[...])

    xz_ref[...] = xpart(kz_ref, kbz_ref, bz_ref)
    xr_ref[...] = xpart(kr_ref, kbr_ref, br_ref)
    xh_ref[...] = xpart(kh_ref, kbh_ref, bxh_ref)

    rz = rz_ref[...]
    rr = rr_ref[...]
    rh = rh_ref[...]
    brh = brh_ref[...]

    def step(t, h):
        row = pl.ds(8 * t, 8)
        z = jax.nn.sigmoid(xz_ref[row, :] +
                           jnp.dot(h, rz, preferred_element_type=jnp.float32))
        r = jax.nn.sigmoid(xr_ref[row, :] +
                           jnp.dot(h, rr, preferred_element_type=jnp.float32))
        hh = jnp.tanh(xh_ref[row, :] + r * (
            jnp.dot(h, rh, preferred_element_type=jnp.float32) + brh))
        h_new = z * h + (1.0 - z) * hh
        out_ref[row, :] = h_new
        return h_new

    jax.lax.fori_loop(0, N, step, jnp.zeros((8, ATOM_DIM), jnp.float32))


def kernel(atom_features, bond_features, connectivity, bond_transform,
           gru_kernel, gru_recurrent_kernel, gru_bias):
    src = connectivity[:, :, 0].reshape(B, E, 1)
    tgt = connectivity[:, :, 1].reshape(B, 1, E)

    agg16 = pl.pallas_call(
        _scatter_body,
        grid=(B,),
        in_specs=[
            pl.BlockSpec((None, N, ATOM_DIM), lambda b: (b, 0, 0)),
            pl.BlockSpec((None, E, 1), lambda b: (b, 0, 0)),
            pl.BlockSpec((None, 1, E), lambda b: (b, 0, 0)),
            pl.BlockSpec((None, E, BOND_DIM), lambda b: (b, 0, 0)),
        ],
        out_specs=pl.BlockSpec((None, N, BOND_DIM), lambda b: (b, 0, 0)),
        out_shape=jax.ShapeDtypeStruct((B, N, BOND_DIM), jnp.float32),
    )(atom_features, src, tgt, bond_features)

    atom_t = jnp.zeros((N, 8, ATOM_DIM), jnp.float32).at[:, :B].set(
        atom_features.transpose(1, 0, 2)).reshape(N * 8, ATOM_DIM)
    agg_t = jnp.zeros((N, 8, BOND_DIM), jnp.float32).at[:, :B].set(
        agg16.transpose(1, 0, 2)).reshape(N * 8, BOND_DIM)

    k_top, k_bot = gru_kernel[:ATOM_DIM], gru_kernel[ATOM_DIM:]
    kz, kr, kh = (k_top[:, :64], k_top[:, 64:128], k_top[:, 128:])
    kbz, kbr, kbh = (k_bot[:, :64], k_bot[:, 64:128], k_bot[:, 128:])
    rz, rr, rh = (gru_recurrent_kernel[:, :64],
                  gru_recurrent_kernel[:, 64:128],
                  gru_recurrent_kernel[:, 128:])
    bi, br = gru_bias[0], gru_bias[1]
    bz = (bi[:64] + br[:64]).reshape(1, 64)
    brg = (bi[64:128] + br[64:128]).reshape(1, 64)
    bxh = bi[128:].reshape(1, 64)
    brh = br[128:].reshape(1, 64)

    out_t = pl.pallas_call(
        _gru_body,
        out_shape=jax.ShapeDtypeStruct((N * 8, ATOM_DIM), jnp.float32),
        scratch_shapes=[pltpu.VMEM((N * 8, ATOM_DIM), jnp.float32)] * 3,
    )(atom_t, agg_t, bond_transform, kz, kr, kh, kbz, kbr, kbh,
      rz, rr, rh, bz, brg, bxh, brh)

    return out_t.reshape(N, 8, ATOM_DIM)[:, :B].transpose(1, 0, 2)
